# baseline (speedup 1.0000x reference)
import functools

import jax
import jax.numpy as jnp
from jax import lax
from jax.experimental import pallas as pl
from jax.experimental.pallas import tpu as pltpu
from jax.experimental.pallas import tpu_sc as plsc

EPS = 1e-5
LANES = 16
NUM_WORKERS = 32
G = 4
BATCH = G * LANES
P = 64
NS = 64
OUT_P = P + NS


def _compute_batch(lt, wt, ot, cdfts, binsts, belowts, markers, lane, epoch,
                   pre_merge=None):
    lanes = [lane + (LANES * g) for g in range(G)]
    zero_f = jnp.zeros((LANES,), jnp.float32)
    zero_i = jnp.zeros((LANES,), jnp.int32)

    for g in range(G):
        cdfts[g][0] = zero_f

    def cdf_body(k, carry):
        runs, prevs = carry
        kv = jnp.full((LANES,), k, jnp.int32)
        new_runs, new_prevs = [], []
        for g in range(G):
            w = plsc.load_gather(wt, [lanes[g], kv + 1])
            lcol = plsc.load_gather(lt, [lanes[g], kv + 1])
            r = runs[g] + (w + EPS)
            cdfts[g][k + 1] = r
            binsts[g][k] = 0.5 * (prevs[g] + lcol)
            new_runs.append(r)
            new_prevs.append(lcol)
        return tuple(new_runs), tuple(new_prevs)

    prev0 = tuple(plsc.load_gather(lt, [lanes[g], zero_i]) for g in range(G))
    totals, prevs = plsc.parallel_loop(
        0, P - 2, 1, unroll=4, carry=((zero_f,) * G, prev0))(cdf_body)
    kv62 = jnp.full((LANES,), P - 1, jnp.int32)
    for g in range(G):
        lcol = plsc.load_gather(lt, [lanes[g], kv62])
        binsts[g][P - 2] = 0.5 * (prevs[g] + lcol)

    invs = [(NS - 1.0) / totals[g] for g in range(G)]

    def init_body(j, c):
        for g in range(G):
            belowts[g][j] = zero_i
        return c

    plsc.parallel_loop(0, NS, 1, unroll=4, carry=jnp.int32(0))(init_body)

    def ceil_pos(x):
        i = x.astype(jnp.int32)
        return jnp.where(i.astype(jnp.float32) < x, i + 1, i)

    def pos_body(k, pcurs):
        kv = jnp.full((LANES,), k, jnp.int32)
        new = []
        for g in range(G):
            pnext = ceil_pos(cdfts[g][k + 1] * invs[g])
            p = jnp.clip(pcurs[g], 0, NS - 1)
            plsc.store_scatter(belowts[g], [p, lane], kv, mask=pnext > pcurs[g])
            new.append(pnext)
        return tuple(new)

    plast = plsc.parallel_loop(
        0, P - 2, 1, unroll=4, carry=(zero_i,) * G)(pos_body)
    kv62 = jnp.full((LANES,), P - 2, jnp.int32)
    for g in range(G):
        plsc.store_scatter(belowts[g], [jnp.clip(plast[g], 0, NS - 1), lane],
                           kv62)

    if pre_merge is not None:
        pre_merge()

    def j_body(j, runbs):
        uf = lax.convert_element_type(j, jnp.float32) * (1.0 / (NS - 1.0))
        jv = jnp.full((LANES,), j, jnp.int32)
        out = []
        for g in range(G):
            runb = jnp.maximum(runbs[g], belowts[g][j])
            bi = runb
            ai = jnp.minimum(bi + 1, P - 2)
            cb = plsc.load_gather(cdfts[g], [bi, lane])
            ca = plsc.load_gather(cdfts[g], [ai, lane])
            bb = plsc.load_gather(binsts[g], [bi, lane])
            ba = plsc.load_gather(binsts[g], [ai, lane])
            lb0 = plsc.load_gather(lt, [lanes[g], bi])
            lb1 = plsc.load_gather(lt, [lanes[g], bi + 1])
            lb2 = plsc.load_gather(lt, [lanes[g], jnp.minimum(bi + 2, P - 1)])
            u = uf * totals[g]
            den = ca - cb
            safe = jnp.where(den < EPS * totals[g], totals[g], den)
            t = (u - cb) / safe
            z = bb + t * (ba - bb)
            cnt = bi + jnp.where(lb0 <= z, 1, 0) + jnp.where(lb1 <= z, 1, 0)
            cnt = cnt + jnp.where((bi + 2 <= P - 1) & (lb2 <= z), 1, 0)
            rank = jv + cnt
            plsc.store_scatter(ot, [lanes[g], rank], z)
            plsc.store_scatter(markers[g], [rank, lane], epoch)
            out.append(runb)
        return tuple(out)

    plsc.parallel_loop(0, NS, 1, unroll=4, carry=(zero_i,) * G)(j_body)

    def h_body(i, hcnts):
        iv = jnp.full((LANES,), i, jnp.int32)
        new = []
        for g in range(G):
            is_hole = markers[g][i] != epoch
            val = plsc.load_gather(lt, [lanes[g], jnp.minimum(hcnts[g], P - 1)])
            plsc.store_scatter(ot, [lanes[g], iv], val, mask=is_hole)
            new.append(hcnts[g] + jnp.where(is_hole, 1, 0))
        return tuple(new)

    plsc.parallel_loop(0, OUT_P, 1, unroll=8, carry=(zero_i,) * G)(h_body)


def _refine_body(lengths_hbm, weights_hbm, out_hbm, lt0, lt1, wt0, wt1,
                 ot,
                 cdft0, cdft1, cdft2, cdft3,
                 binst0, binst1, binst2, binst3,
                 belowt0, belowt1, belowt2, belowt3,
                 mk0, mk1, mk2, mk3,
                 sl0, sl1, sw0, sw1, so):
    cdfts = [cdft0, cdft1, cdft2, cdft3]
    binsts = [binst0, binst1, binst2, binst3]
    belowts = [belowt0, belowt1, belowt2, belowt3]
    markers = [mk0, mk1, mk2, mk3]
    num_rays = lengths_hbm.shape[0]
    rays_per_worker = num_rays // NUM_WORKERS
    nb = rays_per_worker // BATCH

    wid = lax.axis_index("s") * 2 + lax.axis_index("c")
    w_base = wid * rays_per_worker
    lane = lax.iota(jnp.int32, LANES)

    def in_l(slot_ref, sem, b):
        return pltpu.make_async_copy(
            lengths_hbm.at[pl.ds(w_base + b * BATCH, BATCH)], slot_ref, sem)

    def in_w(slot_ref, sem, b):
        return pltpu.make_async_copy(
            weights_hbm.at[pl.ds(w_base + b * BATCH, BATCH)], slot_ref, sem)

    def out_c(slot_ref, sem, b):
        return pltpu.make_async_copy(
            slot_ref, out_hbm.at[pl.ds(w_base + b * BATCH, BATCH)], sem)

    in_l(lt0, sl0, 0).start()
    in_w(wt0, sw0, 0).start()
    zero_i = jnp.zeros((LANES,), jnp.int32)

    def mz_body(i, c):
        for g in range(G):
            markers[g][i] = zero_i
        return c

    lax.fori_loop(0, OUT_P, mz_body, 0)

    def wait_out():
        out_c(ot, so, 0).wait()

    def pair_body(b2, c):
        e = b2 * 2
        o = e + 1
        in_l(lt0, sl0, e).wait()
        in_w(wt0, sw0, e).wait()
        in_l(lt1, sl1, o).start()
        in_w(wt1, sw1, o).start()

        def pre_merge_e():
            @pl.when(b2 > 0)
            def _():
                wait_out()

        _compute_batch(lt0, wt0, ot, cdfts, binsts, belowts, markers, lane,
                       jnp.full((LANES,), e + 1, jnp.int32),
                       pre_merge=pre_merge_e)
        out_c(ot, so, e).start()

        in_l(lt1, sl1, o).wait()
        in_w(wt1, sw1, o).wait()

        @pl.when(b2 < nb // 2 - 1)
        def _():
            in_l(lt0, sl0, o + 1).start()
            in_w(wt0, sw0, o + 1).start()

        _compute_batch(lt1, wt1, ot, cdfts, binsts, belowts, markers, lane,
                       jnp.full((LANES,), o + 1, jnp.int32),
                       pre_merge=wait_out)
        out_c(ot, so, o).start()
        return c

    lax.fori_loop(0, nb // 2, pair_body, 0)
    wait_out()


@jax.jit
def _refine(lengths2d, weights2d):
    num_rays = lengths2d.shape[0]
    mesh = plsc.VectorSubcoreMesh(core_axis_name="c", subcore_axis_name="s")
    return pl.kernel(
        _refine_body,
        out_type=jax.ShapeDtypeStruct((num_rays, OUT_P), jnp.float32),
        mesh=mesh,
        compiler_params=pltpu.CompilerParams(
            needs_layout_passes=False, use_tc_tiling_on_sc=False),
        scratch_types=[
            pltpu.VMEM((BATCH, P), jnp.float32),
            pltpu.VMEM((BATCH, P), jnp.float32),
            pltpu.VMEM((BATCH, P), jnp.float32),
            pltpu.VMEM((BATCH, P), jnp.float32),
            pltpu.VMEM((BATCH, OUT_P), jnp.float32),
            *[pltpu.VMEM((P - 1, LANES), jnp.float32) for _ in range(G)],
            *[pltpu.VMEM((P - 1, LANES), jnp.float32) for _ in range(G)],
            *[pltpu.VMEM((NS, LANES), jnp.int32) for _ in range(G)],
            *[pltpu.VMEM((OUT_P, LANES), jnp.int32) for _ in range(G)],
            pltpu.SemaphoreType.DMA,
            pltpu.SemaphoreType.DMA,
            pltpu.SemaphoreType.DMA,
            pltpu.SemaphoreType.DMA,
            pltpu.SemaphoreType.DMA,
        ],
    )(lengths2d, weights2d)


def kernel(origins, directions, lengths, xys, ray_weights):
    b, r, p = lengths.shape
    z_out = _refine(lengths.reshape(b * r, p), ray_weights.reshape(b * r, p))
    return (origins, directions, z_out.reshape(b, r, OUT_P), xys)


# speedup vs baseline: 1.4901x; 1.4901x over previous
import functools

import jax
import jax.numpy as jnp
from jax import lax
from jax.experimental import pallas as pl
from jax.experimental.pallas import tpu as pltpu
from jax.experimental.pallas import tpu_sc as plsc

EPS = 1e-5
LANES = 16
NUM_WORKERS = 32
G = 4
BATCH = G * LANES
P = 64
NS = 64
OUT_P = P + NS


def _compute_batch(lt, wt, ot, cdfts, binsts, belowts, markers, lane, epoch,
                   pre_merge=None):
    lanes = [lane + (LANES * g) for g in range(G)]
    zero_f = jnp.zeros((LANES,), jnp.float32)
    zero_i = jnp.zeros((LANES,), jnp.int32)

    for g in range(G):
        cdfts[g][0] = zero_f

    def cdf_body(k, carry):
        runs, prevs = carry
        kv = jnp.full((LANES,), k, jnp.int32)
        new_runs, new_prevs = [], []
        for g in range(G):
            w = plsc.load_gather(wt, [lanes[g], kv + 1])
            lcol = plsc.load_gather(lt, [lanes[g], kv + 1])
            r = runs[g] + (w + EPS)
            cdfts[g][k + 1] = r
            binsts[g][k] = 0.5 * (prevs[g] + lcol)
            new_runs.append(r)
            new_prevs.append(lcol)
        return tuple(new_runs), tuple(new_prevs)

    prev0 = tuple(plsc.load_gather(lt, [lanes[g], zero_i]) for g in range(G))
    totals, prevs = plsc.parallel_loop(
        0, P - 2, 1, unroll=2, carry=((zero_f,) * G, prev0))(cdf_body)
    kv62 = jnp.full((LANES,), P - 1, jnp.int32)
    for g in range(G):
        lcol = plsc.load_gather(lt, [lanes[g], kv62])
        binsts[g][P - 2] = 0.5 * (prevs[g] + lcol)

    invs = [(NS - 1.0) / totals[g] for g in range(G)]

    def init_body(j, c):
        for g in range(G):
            belowts[g][j] = zero_i
        return c

    plsc.parallel_loop(0, NS, 1, unroll=4, carry=jnp.int32(0))(init_body)

    def ceil_pos(x):
        i = x.astype(jnp.int32)
        return jnp.where(i.astype(jnp.float32) < x, i + 1, i)

    def pos_body(k, pcurs):
        kv = jnp.full((LANES,), k, jnp.int32)
        new = []
        for g in range(G):
            pnext = ceil_pos(cdfts[g][k + 1] * invs[g])
            p = jnp.clip(pcurs[g], 0, NS - 1)
            plsc.store_scatter(belowts[g], [p, lane], kv, mask=pnext > pcurs[g])
            new.append(pnext)
        return tuple(new)

    plast = plsc.parallel_loop(
        0, P - 2, 1, unroll=2, carry=(zero_i,) * G)(pos_body)
    kv62 = jnp.full((LANES,), P - 2, jnp.int32)
    for g in range(G):
        plsc.store_scatter(belowts[g], [jnp.clip(plast[g], 0, NS - 1), lane],
                           kv62)

    if pre_merge is not None:
        pre_merge()

    def j_body(j, runbs):
        uf = lax.convert_element_type(j, jnp.float32) * (1.0 / (NS - 1.0))
        jv = jnp.full((LANES,), j, jnp.int32)
        out = []
        for g in range(G):
            runb = jnp.maximum(runbs[g], belowts[g][j])
            bi = runb
            ai = jnp.minimum(bi + 1, P - 2)
            cb = plsc.load_gather(cdfts[g], [bi, lane])
            ca = plsc.load_gather(cdfts[g], [ai, lane])
            bb = plsc.load_gather(binsts[g], [bi, lane])
            ba = plsc.load_gather(binsts[g], [ai, lane])
            lb1 = plsc.load_gather(lt, [lanes[g], bi + 1])
            u = uf * totals[g]
            den = ca - cb
            safe = jnp.where(den < EPS * totals[g], totals[g], den)
            t = (u - cb) / safe
            z = bb + t * (ba - bb)
            cnt = bi + 1 + jnp.where(lb1 <= z, 1, 0)
            rank = jv + cnt
            plsc.store_scatter(ot, [lanes[g], rank], z)
            plsc.store_scatter(markers[g], [rank, lane], epoch)
            out.append(runb)
        return tuple(out)

    plsc.parallel_loop(0, NS, 1, unroll=2, carry=(zero_i,) * G)(j_body)

    def h_body(i, hcnts):
        iv = jnp.full((LANES,), i, jnp.int32)
        new = []
        for g in range(G):
            is_hole = markers[g][i] != epoch
            val = plsc.load_gather(lt, [lanes[g], jnp.minimum(hcnts[g], P - 1)])
            plsc.store_scatter(ot, [lanes[g], iv], val, mask=is_hole)
            new.append(hcnts[g] + jnp.where(is_hole, 1, 0))
        return tuple(new)

    plsc.parallel_loop(0, OUT_P, 1, unroll=4, carry=(zero_i,) * G)(h_body)


def _refine_body(lengths_hbm, weights_hbm, out_hbm, lt0, lt1, wt0, wt1,
                 ot,
                 cdft0, cdft1, cdft2, cdft3,
                 binst0, binst1, binst2, binst3,
                 belowt0, belowt1, belowt2, belowt3,
                 mk0, mk1, mk2, mk3,
                 sl0, sl1, sw0, sw1, so):
    cdfts = [cdft0, cdft1, cdft2, cdft3]
    binsts = [binst0, binst1, binst2, binst3]
    belowts = [belowt0, belowt1, belowt2, belowt3]
    markers = [mk0, mk1, mk2, mk3]
    num_rays = lengths_hbm.shape[0]
    rays_per_worker = num_rays // NUM_WORKERS
    nb = rays_per_worker // BATCH

    wid = lax.axis_index("s") * 2 + lax.axis_index("c")
    w_base = wid * rays_per_worker
    lane = lax.iota(jnp.int32, LANES)

    def in_l(slot_ref, sem, b):
        return pltpu.make_async_copy(
            lengths_hbm.at[pl.ds(w_base + b * BATCH, BATCH)], slot_ref, sem)

    def in_w(slot_ref, sem, b):
        return pltpu.make_async_copy(
            weights_hbm.at[pl.ds(w_base + b * BATCH, BATCH)], slot_ref, sem)

    def out_c(slot_ref, sem, b):
        return pltpu.make_async_copy(
            slot_ref, out_hbm.at[pl.ds(w_base + b * BATCH, BATCH)], sem)

    in_l(lt0, sl0, 0).start()
    in_w(wt0, sw0, 0).start()
    zero_i = jnp.zeros((LANES,), jnp.int32)

    def mz_body(i, c):
        for g in range(G):
            markers[g][i] = zero_i
        return c

    lax.fori_loop(0, OUT_P, mz_body, 0)

    def wait_out():
        out_c(ot, so, 0).wait()

    def pair_body(b2, c):
        e = b2 * 2
        o = e + 1
        in_l(lt0, sl0, e).wait()
        in_w(wt0, sw0, e).wait()
        in_l(lt1, sl1, o).start()
        in_w(wt1, sw1, o).start()

        def pre_merge_e():
            @pl.when(b2 > 0)
            def _():
                wait_out()

        _compute_batch(lt0, wt0, ot, cdfts, binsts, belowts, markers, lane,
                       jnp.full((LANES,), e + 1, jnp.int32),
                       pre_merge=pre_merge_e)
        out_c(ot, so, e).start()

        in_l(lt1, sl1, o).wait()
        in_w(wt1, sw1, o).wait()

        @pl.when(b2 < nb // 2 - 1)
        def _():
            in_l(lt0, sl0, o + 1).start()
            in_w(wt0, sw0, o + 1).start()

        _compute_batch(lt1, wt1, ot, cdfts, binsts, belowts, markers, lane,
                       jnp.full((LANES,), o + 1, jnp.int32),
                       pre_merge=wait_out)
        out_c(ot, so, o).start()
        return c

    lax.fori_loop(0, nb // 2, pair_body, 0)
    wait_out()


@jax.jit
def _refine(lengths2d, weights2d):
    num_rays = lengths2d.shape[0]
    mesh = plsc.VectorSubcoreMesh(core_axis_name="c", subcore_axis_name="s")
    return pl.kernel(
        _refine_body,
        out_type=jax.ShapeDtypeStruct((num_rays, OUT_P), jnp.float32),
        mesh=mesh,
        compiler_params=pltpu.CompilerParams(
            needs_layout_passes=False, use_tc_tiling_on_sc=False),
        scratch_types=[
            pltpu.VMEM((BATCH, P), jnp.float32),
            pltpu.VMEM((BATCH, P), jnp.float32),
            pltpu.VMEM((BATCH, P), jnp.float32),
            pltpu.VMEM((BATCH, P), jnp.float32),
            pltpu.VMEM((BATCH, OUT_P), jnp.float32),
            *[pltpu.VMEM((P - 1, LANES), jnp.float32) for _ in range(G)],
            *[pltpu.VMEM((P - 1, LANES), jnp.float32) for _ in range(G)],
            *[pltpu.VMEM((NS, LANES), jnp.int32) for _ in range(G)],
            *[pltpu.VMEM((OUT_P, LANES), jnp.int32) for _ in range(G)],
            pltpu.SemaphoreType.DMA,
            pltpu.SemaphoreType.DMA,
            pltpu.SemaphoreType.DMA,
            pltpu.SemaphoreType.DMA,
            pltpu.SemaphoreType.DMA,
        ],
    )(lengths2d, weights2d)


def kernel(origins, directions, lengths, xys, ray_weights):
    b, r, p = lengths.shape
    z_out = _refine(lengths.reshape(b * r, p), ray_weights.reshape(b * r, p))
    return (origins, directions, z_out.reshape(b, r, OUT_P), xys)
